# baseline (speedup 1.0000x reference)
import functools

import jax
import jax.numpy as jnp
from jax import lax
from jax.experimental import pallas as pl
from jax.experimental.pallas import tpu as pltpu
from jax.experimental.pallas import tpu_sc as plsc

N_POS = 8192
D_MODEL = 1024
N_CORES = 2
N_SUBCORES = 16
N_WORKERS = N_CORES * N_SUBCORES
ROWS_PER_W = N_POS // N_WORKERS


def _copy_body(table_hbm, out_hbm, sem):
    wid = lax.axis_index("s") * N_CORES + lax.axis_index("c")
    base = wid * ROWS_PER_W
    pltpu.async_copy(
        table_hbm.at[pl.ds(base, ROWS_PER_W)],
        out_hbm.at[pl.ds(base, ROWS_PER_W)],
        sem,
    ).wait()


@functools.partial(
    pl.kernel,
    mesh=plsc.VectorSubcoreMesh(core_axis_name="c", subcore_axis_name="s"),
    out_type=jax.ShapeDtypeStruct((N_POS, D_MODEL), jnp.float32),
    scratch_types=[pltpu.SemaphoreType.DMA],
)
def _sc_copy(table_hbm, out_hbm, sem):
    _copy_body(table_hbm, out_hbm, sem)


def kernel(x, positional_embedding):
    del x
    return _sc_copy(positional_embedding)


# speedup vs baseline: 24.3660x; 24.3660x over previous
import functools

import jax
import jax.numpy as jnp
from jax import lax
from jax.experimental import pallas as pl
from jax.experimental.pallas import tpu as pltpu
from jax.experimental.pallas import tpu_sc as plsc

N_POS = 8192
D_MODEL = 1024
N_CORES = 2
N_SUBCORES = 16
N_WORKERS = N_CORES * N_SUBCORES
ROWS_PER_W = N_POS // N_WORKERS


CHUNK = 32
N_CHUNKS = ROWS_PER_W // CHUNK


@functools.partial(
    pl.kernel,
    mesh=plsc.VectorSubcoreMesh(core_axis_name="c", subcore_axis_name="s"),
    out_type=jax.ShapeDtypeStruct((N_POS, D_MODEL), jnp.float32),
    scratch_types=[
        pltpu.VMEM((2, CHUNK, D_MODEL), jnp.float32),
        pltpu.SemaphoreType.DMA,
        pltpu.SemaphoreType.DMA,
        pltpu.SemaphoreType.DMA,
        pltpu.SemaphoreType.DMA,
    ],
)
def _sc_copy(table_hbm, out_hbm, buf, rsem0, rsem1, wsem0, wsem1):
    wid = lax.axis_index("s") * N_CORES + lax.axis_index("c")
    base = wid * ROWS_PER_W
    rsems = (rsem0, rsem1)
    wsems = (wsem0, wsem1)

    def rd(i, b):
        return pltpu.make_async_copy(
            table_hbm.at[pl.ds(base + i * CHUNK, CHUNK)], buf.at[b], rsems[b])

    def wr(i, b):
        return pltpu.make_async_copy(
            buf.at[b], out_hbm.at[pl.ds(base + i * CHUNK, CHUNK)], wsems[b])

    rd(0, 0).start()
    rd(1, 1).start()
    for i in range(N_CHUNKS):
        b = i % 2
        rd(i, b).wait()
        wr(i, b).start()
        if i + 2 < N_CHUNKS:
            wr(i, b).wait()
            rd(i + 2, b).start()
    wr(N_CHUNKS - 2, N_CHUNKS % 2).wait()
    wr(N_CHUNKS - 1, (N_CHUNKS - 1) % 2).wait()


def kernel(x, positional_embedding):
    del x
    return _sc_copy(positional_embedding)
